# baseline (speedup 1.0000x reference)
import functools
import math

import jax
import jax.numpy as jnp
from jax import lax
from jax.experimental import pallas as pl
from jax.experimental.pallas import tpu as pltpu
from jax.experimental.pallas import tpu_sc as plsc

_NUM_UNITS = 1024
_SCALE = float(_NUM_UNITS) ** 0.5
_BT = 256


def _pe_tile(o_ref, sinx_ref, cosx_ref, *, bt, num_units):
    t = pl.program_id(0)
    col = jax.lax.broadcasted_iota(jnp.int32, (1, num_units), 1)
    inv = jnp.exp(
        col.astype(jnp.float32) * (-2.0 * math.log(10000.0) / float(num_units))
    )

    @pl.when(t == 0)
    def _():
        phase = (col % 2).astype(jnp.float32) * (math.pi / 2.0)
        r8 = jax.lax.broadcasted_iota(jnp.int32, (8, num_units), 0).astype(
            jnp.float32
        )
        x8 = r8 * inv + phase
        s = jnp.sin(x8)
        c = jnp.cos(x8)
        k = 8
        while k < bt:
            sd = jnp.sin(float(k) * inv)
            cd = jnp.cos(float(k) * inv)
            s, c = (
                jnp.concatenate([s, s * cd + c * sd], axis=0),
                jnp.concatenate([c, c * cd - s * sd], axis=0),
            )
            k *= 2
        sinx_ref[...] = s
        cosx_ref[...] = c

    p = (t * bt).astype(jnp.float32) * inv
    sp = jnp.sin(p) * _SCALE
    cp = jnp.cos(p) * _SCALE
    val = sp * cosx_ref[...] + cp * sinx_ref[...]
    o_ref[...] = val

    @pl.when(t == 0)
    def _():
        o_ref[0:1, :] = jnp.zeros_like(o_ref[0:1, :])


def _build_table(t_len, num_units, bt):
    return pl.pallas_call(
        functools.partial(_pe_tile, bt=bt, num_units=num_units),
        grid=(t_len // bt,),
        out_specs=pl.BlockSpec((bt, num_units), lambda g: (g, 0)),
        out_shape=jax.ShapeDtypeStruct((t_len, num_units), jnp.float32),
        scratch_shapes=[
            pltpu.VMEM((bt, num_units), jnp.float32),
            pltpu.VMEM((bt, num_units), jnp.float32),
        ],
    )()


def _make_sc_broadcast(n, t_len, num_units):
    info = plsc.get_sparse_core_info()
    nc, ns = info.num_cores, info.num_subcores
    nw = nc * ns
    rows_per_w = t_len // nw
    chunk = min(rows_per_w, 64)
    n_chunks = rows_per_w // chunk
    mesh = plsc.VectorSubcoreMesh(core_axis_name="c", subcore_axis_name="s")

    @functools.partial(
        pl.kernel,
        mesh=mesh,
        out_type=jax.ShapeDtypeStruct((n, t_len, num_units), jnp.float32),
        scratch_types=[
            pltpu.VMEM((chunk, num_units), jnp.float32),
            pltpu.SemaphoreType.DMA,
        ],
    )
    def sc_broadcast(table_hbm, out_hbm, buf, sem):
        wid = lax.axis_index("s") * nc + lax.axis_index("c")
        base = wid * rows_per_w
        for j in range(n_chunks):
            r0 = base + j * chunk
            pltpu.sync_copy(table_hbm.at[pl.ds(r0, chunk)], buf)
            copies = [
                pltpu.async_copy(
                    buf, out_hbm.at[i, pl.ds(r0, chunk)], sem
                )
                for i in range(n)
            ]
            for cp_ in copies:
                cp_.wait()

    return sc_broadcast


def kernel(inputs):
    n, t_len = inputs.shape
    num_units = _NUM_UNITS
    table = _build_table(t_len, num_units, _BT)
    return _make_sc_broadcast(n, t_len, num_units)(table)


# speedup vs baseline: 2.4498x; 2.4498x over previous
import functools
import math

import jax
import jax.numpy as jnp
from jax.experimental import pallas as pl
from jax.experimental.pallas import tpu as pltpu

_NUM_UNITS = 1024
_SCALE = float(_NUM_UNITS) ** 0.5
_BT = 256


def _pe_tile(o_ref, sinx_ref, cosx_ref, *, bt, num_units):
    t = pl.program_id(0)
    col = jax.lax.broadcasted_iota(jnp.int32, (1, num_units), 1)
    inv = jnp.exp(
        col.astype(jnp.float32) * (-2.0 * math.log(10000.0) / float(num_units))
    )

    @pl.when(t == 0)
    def _():
        phase = (col % 2).astype(jnp.float32) * (math.pi / 2.0)
        r8 = jax.lax.broadcasted_iota(jnp.int32, (8, num_units), 0).astype(
            jnp.float32
        )
        x8 = r8 * inv + phase
        s = jnp.sin(x8)
        c = jnp.cos(x8)
        k = 8
        while k < bt:
            sd = jnp.sin(float(k) * inv)
            cd = jnp.cos(float(k) * inv)
            s, c = (
                jnp.concatenate([s, s * cd + c * sd], axis=0),
                jnp.concatenate([c, c * cd - s * sd], axis=0),
            )
            k *= 2
        sinx_ref[...] = s
        cosx_ref[...] = c

    p = (t * bt).astype(jnp.float32) * inv
    sp = jnp.sin(p) * _SCALE
    cp = jnp.cos(p) * _SCALE
    val = sp * cosx_ref[...] + cp * sinx_ref[...]
    o_ref[...] = jnp.broadcast_to(val[None], o_ref.shape)

    @pl.when(t == 0)
    def _():
        o_ref[:, 0:1, :] = jnp.zeros_like(o_ref[:, 0:1, :])


def kernel(inputs):
    n, t_len = inputs.shape
    num_units = _NUM_UNITS
    bt = _BT
    grid = (t_len // bt,)
    out = pl.pallas_call(
        functools.partial(_pe_tile, bt=bt, num_units=num_units),
        grid=grid,
        out_specs=pl.BlockSpec((n, bt, num_units), lambda g: (0, g, 0)),
        out_shape=jax.ShapeDtypeStruct((n, t_len, num_units), jnp.float32),
        scratch_shapes=[
            pltpu.VMEM((bt, num_units), jnp.float32),
            pltpu.VMEM((bt, num_units), jnp.float32),
        ],
    )()
    return out
